# baseline (speedup 1.0000x reference)
import jax
import jax.numpy as jnp
from jax.experimental import pallas as pl
from jax.experimental.pallas import tpu as pltpu

_TOPK = 6
_NE = 64
_BLK = 2048
_BIAS = 1 << 29


def _gate_body_t(x_ref, W_ref, w_ref, i_ref):
    s = jax.lax.dot_general(
        W_ref[...], x_ref[...], (((1,), (1,)), ((), ())),
        preferred_element_type=jnp.float32)
    m = jnp.max(s, axis=0, keepdims=True)
    e = jnp.exp(s - m)
    p = e / jnp.sum(e, axis=0, keepdims=True)
    sub = jax.lax.broadcasted_iota(jnp.int32, s.shape, 0)
    pb = jax.lax.bitcast_convert_type(p, jnp.int32)
    key = jax.lax.bitcast_convert_type(
        ((pb & -_NE) | (_NE - 1 - sub)) + _BIAS, jnp.float32)
    picks = []
    for _ in range(_TOPK):
        km = jnp.max(key, axis=0, keepdims=True)
        picks.append(km)
        key = jnp.where(key == km, -1.0, key)
    top = jax.lax.bitcast_convert_type(
        jnp.concatenate(picks, axis=0), jnp.int32) - _BIAS
    w_ref[...] = jax.lax.bitcast_convert_type(top & -_NE, jnp.float32)
    i_ref[...] = _NE - 1 - (top & (_NE - 1))


def kernel(x, W):
    n, d = x.shape
    grid = (n // _BLK,)
    w_t, i_t = pl.pallas_call(
        _gate_body_t,
        grid=grid,
        in_specs=[
            pl.BlockSpec((_BLK, d), lambda i: (i, 0)),
            pl.BlockSpec((_NE, d), lambda i: (0, 0)),
        ],
        out_specs=[
            pl.BlockSpec((_TOPK, _BLK), lambda i: (0, i)),
            pl.BlockSpec((_TOPK, _BLK), lambda i: (0, i)),
        ],
        out_shape=[
            jax.ShapeDtypeStruct((_TOPK, n), jnp.float32),
            jax.ShapeDtypeStruct((_TOPK, n), jnp.int32),
        ],
        compiler_params=pltpu.CompilerParams(
            dimension_semantics=("parallel",),
        ),
    )(x, W)
    return w_t.T, i_t.T
